# Initial kernel scaffold; baseline (speedup 1.0000x reference)
import jax
import jax.numpy as jnp
from jax.experimental import pallas as pl


def kernel(x, edge_index, batch_index, W0, b0, W1, b1, W2, b2, W3, b3, Wout, bout):
    raise NotImplementedError("write your pallas kernel here")


# speedup vs baseline: 3.9288x; 3.9288x over previous
import functools

import jax
import jax.numpy as jnp
from jax import lax
from jax.experimental import pallas as pl
from jax.experimental.pallas import tpu as pltpu
from jax.experimental.pallas import tpu_sc as plsc

N = 100000
E = 1600000
NGRAPH = 1024
EMB = 64
INF = 9

ROW_BLK = 512
NBLK = 196
N_PAD = NBLK * ROW_BLK
ACC_ROWS = 100368
GARBAGE = N_PAD
Z_ROWS = ACC_ROWS // 16
OUT_ROWS = N_PAD // 16

EB = 128
NSUB = 32
NB = 391
E_PAD = NSUB * NB * EB
FP = 4
FW = 16

_MESH = plsc.VectorSubcoreMesh(core_axis_name="c", subcore_axis_name="s")


def _sc_deg_body(dst_hbm, ones_hbm, zero_hbm, o0, o1, acc, didx_v, ones_v):
    c = lax.axis_index("c")
    s = lax.axis_index("s")
    w = c * 16 + s
    pltpu.sync_copy(ones_hbm, ones_v)
    pltpu.sync_copy(zero_hbm, acc.at[pl.ds(s * Z_ROWS, Z_ROWS)])
    plsc.subcore_barrier()

    def bb(b, carry):
        g = (w * NB + b) * EB
        pltpu.sync_copy(dst_hbm.at[pl.ds(g, EB)], didx_v)
        pltpu.sync_copy(ones_v, acc.at[didx_v], add=True)
        return carry

    lax.fori_loop(0, NB, bb, 0)
    plsc.subcore_barrier()

    @pl.when(c == 0)
    def _():
        pltpu.sync_copy(acc.at[pl.ds(s * OUT_ROWS, OUT_ROWS)],
                        o0.at[pl.ds(s * OUT_ROWS, OUT_ROWS)])

    @pl.when(c == 1)
    def _():
        pltpu.sync_copy(acc.at[pl.ds(s * OUT_ROWS, OUT_ROWS)],
                        o1.at[pl.ds(s * OUT_ROWS, OUT_ROWS)])


_SC_PARAMS = pltpu.CompilerParams(use_tc_tiling_on_sc=False)

_sc_deg = pl.kernel(
    _sc_deg_body,
    out_type=[jax.ShapeDtypeStruct((N_PAD, FW), jnp.float32)] * 2,
    mesh=_MESH,
    compiler_params=_SC_PARAMS,
    scratch_types=[
        pltpu.VMEM_SHARED((ACC_ROWS, FW), jnp.float32),
        pltpu.VMEM((EB,), jnp.int32),
        pltpu.VMEM((EB, FW), jnp.float32),
    ],
)


def _sc_agg_body(g0, g1, g2, g3, src_hbm, dst_hbm, zero_hbm,
                 a00, a01, a02, a03, a10, a11, a12, a13,
                 acc, idx_v, didx_v, rows_v, sem):
    c = lax.axis_index("c")
    s = lax.axis_index("s")
    w = c * 16 + s
    gs_list = [g0, g1, g2, g3]
    outs = [[a00, a01, a02, a03], [a10, a11, a12, a13]]
    for p in range(FP):
        pltpu.sync_copy(zero_hbm, acc.at[pl.ds(s * Z_ROWS, Z_ROWS)])
        plsc.subcore_barrier()

        def bb(b, carry, p=p):
            g = (w * NB + b) * EB
            pltpu.sync_copy(src_hbm.at[pl.ds(g, EB)], idx_v)
            pltpu.sync_copy(dst_hbm.at[pl.ds(g, EB)], didx_v)
            pltpu.async_copy(gs_list[p].at[idx_v], rows_v, sem).wait()
            pltpu.sync_copy(rows_v, acc.at[didx_v], add=True)
            return carry

        lax.fori_loop(0, NB, bb, 0)
        plsc.subcore_barrier()

        @pl.when(c == 0)
        def _(p=p):
            pltpu.sync_copy(acc.at[pl.ds(s * OUT_ROWS, OUT_ROWS)],
                            outs[0][p].at[pl.ds(s * OUT_ROWS, OUT_ROWS)])

        @pl.when(c == 1)
        def _(p=p):
            pltpu.sync_copy(acc.at[pl.ds(s * OUT_ROWS, OUT_ROWS)],
                            outs[1][p].at[pl.ds(s * OUT_ROWS, OUT_ROWS)])

        plsc.subcore_barrier()


_sc_agg = pl.kernel(
    _sc_agg_body,
    out_type=[jax.ShapeDtypeStruct((N_PAD, FW), jnp.float32)] * 8,
    mesh=_MESH,
    compiler_params=_SC_PARAMS,
    scratch_types=[
        pltpu.VMEM_SHARED((ACC_ROWS, FW), jnp.float32),
        pltpu.VMEM((EB,), jnp.int32),
        pltpu.VMEM((EB,), jnp.int32),
        pltpu.VMEM((EB, FW), jnp.float32),
        pltpu.SemaphoreType.DMA,
    ],
)


def _first_body(x_ref, d0_ref, d1_ref, w_ref, o0, o1, o2, o3, dinv_ref):
    cnt = d0_ref[:, :1] + d1_ref[:, :1]
    dinv = lax.rsqrt(cnt + 1.0)
    g = jnp.dot(x_ref[...], w_ref[...], preferred_element_type=jnp.float32)
    gs = dinv * g
    for p, o in enumerate((o0, o1, o2, o3)):
        o[...] = gs[:, p * FW:(p + 1) * FW]
    dinv_ref[...] = dinv


_tc_first = pl.pallas_call(
    _first_body,
    grid=(NBLK,),
    in_specs=[
        pl.BlockSpec((ROW_BLK, INF), lambda i: (i, 0)),
        pl.BlockSpec((ROW_BLK, FW), lambda i: (i, 0)),
        pl.BlockSpec((ROW_BLK, FW), lambda i: (i, 0)),
        pl.BlockSpec((INF, EMB), lambda i: (0, 0)),
    ],
    out_specs=[pl.BlockSpec((ROW_BLK, FW), lambda i: (i, 0))] * 4
    + [pl.BlockSpec((ROW_BLK, 1), lambda i: (i, 0))],
    out_shape=[jax.ShapeDtypeStruct((N_PAD, FW), jnp.float32)] * 4
    + [jax.ShapeDtypeStruct((N_PAD, 1), jnp.float32)],
)


def _layer_body(a00, a01, a02, a03, a10, a11, a12, a13,
                g0, g1, g2, g3, dinv_ref, b_ref, w_ref,
                o0, o1, o2, o3):
    A = jnp.concatenate(
        [a00[...] + a10[...], a01[...] + a11[...],
         a02[...] + a12[...], a03[...] + a13[...]], axis=1)
    G = jnp.concatenate([g0[...], g1[...], g2[...], g3[...]], axis=1)
    dinv = dinv_ref[...]
    h = jnp.tanh(dinv * (A + G) + b_ref[...])
    gs = dinv * jnp.dot(h, w_ref[...], preferred_element_type=jnp.float32)
    for p, o in enumerate((o0, o1, o2, o3)):
        o[...] = gs[:, p * FW:(p + 1) * FW]


_tc_layer = pl.pallas_call(
    _layer_body,
    grid=(NBLK,),
    in_specs=[pl.BlockSpec((ROW_BLK, FW), lambda i: (i, 0))] * 12
    + [
        pl.BlockSpec((ROW_BLK, 1), lambda i: (i, 0)),
        pl.BlockSpec((1, EMB), lambda i: (0, 0)),
        pl.BlockSpec((EMB, EMB), lambda i: (0, 0)),
    ],
    out_specs=[pl.BlockSpec((ROW_BLK, FW), lambda i: (i, 0))] * 4,
    out_shape=[jax.ShapeDtypeStruct((N_PAD, FW), jnp.float32)] * 4,
)


def _final_body(a00, a01, a02, a03, a10, a11, a12, a13,
                g0, g1, g2, g3, dinv_ref, b_ref, wout_ref, bout_ref,
                offs_ref, out_ref, hid_ref, h4_ref):
    i = pl.program_id(0)
    A = jnp.concatenate(
        [a00[...] + a10[...], a01[...] + a11[...],
         a02[...] + a12[...], a03[...] + a13[...]], axis=1)
    G = jnp.concatenate([g0[...], g1[...], g2[...], g3[...]], axis=1)
    h = jnp.tanh(dinv_ref[...] * (A + G) + b_ref[...])
    h4_ref[pl.ds(i * ROW_BLK, ROW_BLK), :] = h

    @pl.when(i == NBLK - 1)
    def _():
        def pool(gi, carry):
            s0 = offs_ref[gi]
            e0 = offs_ref[gi + 1]
            p0 = jnp.bitwise_and(s0, -8)

            def cond(c):
                return c[0] < e0

            def bodyw(c):
                p, mx, sm = c
                rows = h4_ref[pl.ds(p, 128), :]
                ids = p + lax.broadcasted_iota(jnp.int32, (128, 1), 0)
                m = (ids >= s0) & (ids < e0)
                mx = jnp.maximum(mx, jnp.where(m, rows, -jnp.inf))
                sm = sm + jnp.where(m, rows, 0.0)
                return (p + 128, mx, sm)

            neg = jnp.full((128, EMB), -jnp.inf, jnp.float32)
            zer = jnp.zeros((128, EMB), jnp.float32)
            _, mx, sm = lax.while_loop(cond, bodyw, (p0, neg, zer))
            gmx = jnp.max(mx, axis=0, keepdims=True)
            gsm = jnp.sum(sm, axis=0, keepdims=True)
            cntf = jnp.maximum((e0 - s0).astype(jnp.float32), 1.0)
            hid_ref[pl.ds(gi, 1), :] = jnp.concatenate([gmx, gsm / cntf],
                                                       axis=1)
            return carry

        lax.fori_loop(0, NGRAPH, pool, 0)
        hid = hid_ref[...]
        out_ref[...] = jnp.dot(hid, wout_ref[...],
                               preferred_element_type=jnp.float32) + bout_ref[...]


_tc_final = pl.pallas_call(
    _final_body,
    grid=(NBLK,),
    in_specs=[pl.BlockSpec((ROW_BLK, FW), lambda i: (i, 0))] * 12
    + [
        pl.BlockSpec((ROW_BLK, 1), lambda i: (i, 0)),
        pl.BlockSpec((1, EMB), lambda i: (0, 0)),
        pl.BlockSpec((2 * EMB, 1), lambda i: (0, 0)),
        pl.BlockSpec((1, 1), lambda i: (0, 0)),
        pl.BlockSpec(memory_space=pltpu.SMEM),
    ],
    out_specs=[
        pl.BlockSpec((NGRAPH, 1), lambda i: (0, 0)),
        pl.BlockSpec((NGRAPH, 2 * EMB), lambda i: (0, 0)),
    ],
    out_shape=[
        jax.ShapeDtypeStruct((NGRAPH, 1), jnp.float32),
        jax.ShapeDtypeStruct((NGRAPH, 2 * EMB), jnp.float32),
    ],
    scratch_shapes=[pltpu.VMEM((N_PAD, EMB), jnp.float32)],
)


@jax.jit
def kernel(x, edge_index, batch_index, W0, b0, W1, b1, W2, b2, W3, b3,
           Wout, bout):
    src = edge_index[0].astype(jnp.int32)
    dst = edge_index[1].astype(jnp.int32)
    batch = batch_index.astype(jnp.int32)

    srcp = jnp.concatenate([src, jnp.zeros((E_PAD - E,), jnp.int32)])
    dstp = jnp.concatenate([dst, jnp.full((E_PAD - E,), GARBAGE, jnp.int32)])
    xp = jnp.concatenate([x, jnp.zeros((N_PAD - N, INF), jnp.float32)])
    z16 = jnp.zeros((Z_ROWS, FW), jnp.float32)
    ones16 = jnp.ones((EB, FW), jnp.float32)
    offs = jnp.searchsorted(
        batch, jnp.arange(NGRAPH + 1, dtype=jnp.int32)).astype(jnp.int32)

    d0, d1 = _sc_deg(dstp, ones16, z16)
    g0, g1, g2, g3, dinv = _tc_first(xp, d0, d1, W0)
    gs = (g0, g1, g2, g3)
    for W, b in ((W1, b0), (W2, b1), (W3, b2)):
        a = _sc_agg(*gs, srcp, dstp, z16)
        gs = _tc_layer(*a, *gs, dinv, b.reshape(1, EMB), W)
    a = _sc_agg(*gs, srcp, dstp, z16)
    out, hidden = _tc_final(*a, *gs, dinv, b3.reshape(1, EMB),
                            Wout, bout.reshape(1, 1), offs)
    return (out, hidden)


# speedup vs baseline: 8.6246x; 2.1952x over previous
import functools

import jax
import jax.numpy as jnp
from jax import lax
from jax.experimental import pallas as pl
from jax.experimental.pallas import tpu as pltpu
from jax.experimental.pallas import tpu_sc as plsc

N = 100000
E = 1600000
NGRAPH = 1024
EMB = 64
INF = 9

ROW_BLK = 512
NBLK = 196
N_PAD = NBLK * ROW_BLK
ACC_ROWS = 100368
GARBAGE = N_PAD
Z_ROWS = ACC_ROWS // 16
OUT_ROWS = N_PAD // 16

EB = 128
NSUB = 32
NB = 392
KQ = 8
NQ = NB // KQ
E_PAD = NSUB * NB * EB
FP = 4
FW = 16

_MESH = plsc.VectorSubcoreMesh(core_axis_name="c", subcore_axis_name="s")


def _sc_deg_body(dst_hbm, ones_hbm, zero_hbm, o0, o1, acc, didx_v, ones_v,
                 sems):
    c = lax.axis_index("c")
    s = lax.axis_index("s")
    w = c * 16 + s
    pltpu.sync_copy(ones_hbm, ones_v)
    pltpu.sync_copy(zero_hbm, acc.at[pl.ds(s * Z_ROWS, Z_ROWS)])
    plsc.subcore_barrier()

    def bb(q, carry):
        r0 = (w * NQ + q) * KQ
        pltpu.sync_copy(dst_hbm.at[pl.ds(r0, KQ)], didx_v)
        sds = [pltpu.async_copy(ones_v, acc.at[didx_v.at[j]], sems, add=True)
               for j in range(KQ)]
        for d in sds:
            d.wait()
        return carry

    lax.fori_loop(0, NQ, bb, 0)
    plsc.subcore_barrier()

    @pl.when(c == 0)
    def _():
        pltpu.sync_copy(acc.at[pl.ds(s * OUT_ROWS, OUT_ROWS)],
                        o0.at[pl.ds(s * OUT_ROWS, OUT_ROWS)])

    @pl.when(c == 1)
    def _():
        pltpu.sync_copy(acc.at[pl.ds(s * OUT_ROWS, OUT_ROWS)],
                        o1.at[pl.ds(s * OUT_ROWS, OUT_ROWS)])


_SC_PARAMS = pltpu.CompilerParams(use_tc_tiling_on_sc=False)

_sc_deg = pl.kernel(
    _sc_deg_body,
    out_type=[jax.ShapeDtypeStruct((N_PAD, FW), jnp.float32)] * 2,
    mesh=_MESH,
    compiler_params=_SC_PARAMS,
    scratch_types=[
        pltpu.VMEM_SHARED((ACC_ROWS, FW), jnp.float32),
        pltpu.VMEM((KQ, EB), jnp.int32),
        pltpu.VMEM((EB, FW), jnp.float32),
        pltpu.SemaphoreType.DMA,
    ],
)


def _sc_agg_body(g0, g1, g2, g3, src_hbm, dst_hbm, zero_hbm,
                 a00, a01, a02, a03, a10, a11, a12, a13,
                 acc, idx_v, didx_v, rows_v, sem, sems):
    c = lax.axis_index("c")
    s = lax.axis_index("s")
    w = c * 16 + s
    gs_list = [g0, g1, g2, g3]
    outs = [[a00, a01, a02, a03], [a10, a11, a12, a13]]
    for p in range(FP):
        pltpu.sync_copy(zero_hbm, acc.at[pl.ds(s * Z_ROWS, Z_ROWS)])
        plsc.subcore_barrier()

        def bb(q, carry, p=p):
            r0 = (w * NQ + q) * KQ
            pltpu.sync_copy(src_hbm.at[pl.ds(r0, KQ)], idx_v)
            pltpu.sync_copy(dst_hbm.at[pl.ds(r0, KQ)], didx_v)
            gds = [pltpu.async_copy(gs_list[p].at[idx_v.at[j]],
                                    rows_v.at[j], sem)
                   for j in range(KQ)]
            for d in gds:
                d.wait()
            sds = [pltpu.async_copy(rows_v.at[j], acc.at[didx_v.at[j]],
                                    sems, add=True)
                   for j in range(KQ)]
            for d in sds:
                d.wait()
            return carry

        lax.fori_loop(0, NQ, bb, 0)
        plsc.subcore_barrier()

        @pl.when(c == 0)
        def _(p=p):
            pltpu.sync_copy(acc.at[pl.ds(s * OUT_ROWS, OUT_ROWS)],
                            outs[0][p].at[pl.ds(s * OUT_ROWS, OUT_ROWS)])

        @pl.when(c == 1)
        def _(p=p):
            pltpu.sync_copy(acc.at[pl.ds(s * OUT_ROWS, OUT_ROWS)],
                            outs[1][p].at[pl.ds(s * OUT_ROWS, OUT_ROWS)])

        plsc.subcore_barrier()


_sc_agg = pl.kernel(
    _sc_agg_body,
    out_type=[jax.ShapeDtypeStruct((N_PAD, FW), jnp.float32)] * 8,
    mesh=_MESH,
    compiler_params=_SC_PARAMS,
    scratch_types=[
        pltpu.VMEM_SHARED((ACC_ROWS, FW), jnp.float32),
        pltpu.VMEM((KQ, EB), jnp.int32),
        pltpu.VMEM((KQ, EB), jnp.int32),
        pltpu.VMEM((KQ, EB, FW), jnp.float32),
        pltpu.SemaphoreType.DMA,
        pltpu.SemaphoreType.DMA,
    ],
)


def _first_body(x_ref, d0_ref, d1_ref, w_ref, o0, o1, o2, o3, dinv_ref):
    cnt = d0_ref[:, :1] + d1_ref[:, :1]
    dinv = lax.rsqrt(cnt + 1.0)
    g = jnp.dot(x_ref[...], w_ref[...], preferred_element_type=jnp.float32)
    gs = dinv * g
    for p, o in enumerate((o0, o1, o2, o3)):
        o[...] = gs[:, p * FW:(p + 1) * FW]
    dinv_ref[...] = dinv


_tc_first = pl.pallas_call(
    _first_body,
    grid=(NBLK,),
    in_specs=[
        pl.BlockSpec((ROW_BLK, INF), lambda i: (i, 0)),
        pl.BlockSpec((ROW_BLK, FW), lambda i: (i, 0)),
        pl.BlockSpec((ROW_BLK, FW), lambda i: (i, 0)),
        pl.BlockSpec((INF, EMB), lambda i: (0, 0)),
    ],
    out_specs=[pl.BlockSpec((ROW_BLK, FW), lambda i: (i, 0))] * 4
    + [pl.BlockSpec((ROW_BLK, 1), lambda i: (i, 0))],
    out_shape=[jax.ShapeDtypeStruct((N_PAD, FW), jnp.float32)] * 4
    + [jax.ShapeDtypeStruct((N_PAD, 1), jnp.float32)],
)


def _layer_body(a00, a01, a02, a03, a10, a11, a12, a13,
                g0, g1, g2, g3, dinv_ref, b_ref, w_ref,
                o0, o1, o2, o3):
    A = jnp.concatenate(
        [a00[...] + a10[...], a01[...] + a11[...],
         a02[...] + a12[...], a03[...] + a13[...]], axis=1)
    G = jnp.concatenate([g0[...], g1[...], g2[...], g3[...]], axis=1)
    dinv = dinv_ref[...]
    h = jnp.tanh(dinv * (A + G) + b_ref[...])
    gs = dinv * jnp.dot(h, w_ref[...], preferred_element_type=jnp.float32)
    for p, o in enumerate((o0, o1, o2, o3)):
        o[...] = gs[:, p * FW:(p + 1) * FW]


_tc_layer = pl.pallas_call(
    _layer_body,
    grid=(NBLK,),
    in_specs=[pl.BlockSpec((ROW_BLK, FW), lambda i: (i, 0))] * 12
    + [
        pl.BlockSpec((ROW_BLK, 1), lambda i: (i, 0)),
        pl.BlockSpec((1, EMB), lambda i: (0, 0)),
        pl.BlockSpec((EMB, EMB), lambda i: (0, 0)),
    ],
    out_specs=[pl.BlockSpec((ROW_BLK, FW), lambda i: (i, 0))] * 4,
    out_shape=[jax.ShapeDtypeStruct((N_PAD, FW), jnp.float32)] * 4,
)


def _final_body(a00, a01, a02, a03, a10, a11, a12, a13,
                g0, g1, g2, g3, dinv_ref, b_ref, wout_ref, bout_ref,
                offs_ref, out_ref, hid_ref, h4_ref):
    i = pl.program_id(0)
    A = jnp.concatenate(
        [a00[...] + a10[...], a01[...] + a11[...],
         a02[...] + a12[...], a03[...] + a13[...]], axis=1)
    G = jnp.concatenate([g0[...], g1[...], g2[...], g3[...]], axis=1)
    h = jnp.tanh(dinv_ref[...] * (A + G) + b_ref[...])
    h4_ref[pl.ds(i * ROW_BLK, ROW_BLK), :] = h

    @pl.when(i == NBLK - 1)
    def _():
        def pool(gi, carry):
            s0 = offs_ref[gi]
            e0 = offs_ref[gi + 1]
            p0 = jnp.bitwise_and(s0, -8)

            def cond(c):
                return c[0] < e0

            def bodyw(c):
                p, mx, sm = c
                rows = h4_ref[pl.ds(p, 128), :]
                ids = p + lax.broadcasted_iota(jnp.int32, (128, 1), 0)
                m = (ids >= s0) & (ids < e0)
                mx = jnp.maximum(mx, jnp.where(m, rows, -jnp.inf))
                sm = sm + jnp.where(m, rows, 0.0)
                return (p + 128, mx, sm)

            neg = jnp.full((128, EMB), -jnp.inf, jnp.float32)
            zer = jnp.zeros((128, EMB), jnp.float32)
            _, mx, sm = lax.while_loop(cond, bodyw, (p0, neg, zer))
            gmx = jnp.max(mx, axis=0, keepdims=True)
            gsm = jnp.sum(sm, axis=0, keepdims=True)
            cntf = jnp.maximum((e0 - s0).astype(jnp.float32), 1.0)
            hid_ref[pl.ds(gi, 1), :] = jnp.concatenate([gmx, gsm / cntf],
                                                       axis=1)
            return carry

        lax.fori_loop(0, NGRAPH, pool, 0)
        hid = hid_ref[...]
        out_ref[...] = jnp.dot(hid, wout_ref[...],
                               preferred_element_type=jnp.float32) + bout_ref[...]


_tc_final = pl.pallas_call(
    _final_body,
    grid=(NBLK,),
    in_specs=[pl.BlockSpec((ROW_BLK, FW), lambda i: (i, 0))] * 12
    + [
        pl.BlockSpec((ROW_BLK, 1), lambda i: (i, 0)),
        pl.BlockSpec((1, EMB), lambda i: (0, 0)),
        pl.BlockSpec((2 * EMB, 1), lambda i: (0, 0)),
        pl.BlockSpec((1, 1), lambda i: (0, 0)),
        pl.BlockSpec(memory_space=pltpu.SMEM),
    ],
    out_specs=[
        pl.BlockSpec((NGRAPH, 1), lambda i: (0, 0)),
        pl.BlockSpec((NGRAPH, 2 * EMB), lambda i: (0, 0)),
    ],
    out_shape=[
        jax.ShapeDtypeStruct((NGRAPH, 1), jnp.float32),
        jax.ShapeDtypeStruct((NGRAPH, 2 * EMB), jnp.float32),
    ],
    scratch_shapes=[pltpu.VMEM((N_PAD, EMB), jnp.float32)],
)


@jax.jit
def kernel(x, edge_index, batch_index, W0, b0, W1, b1, W2, b2, W3, b3,
           Wout, bout):
    src = edge_index[0].astype(jnp.int32)
    dst = edge_index[1].astype(jnp.int32)
    batch = batch_index.astype(jnp.int32)

    srcp = jnp.concatenate(
        [src, jnp.zeros((E_PAD - E,), jnp.int32)]).reshape(E_PAD // EB, EB)
    dstp = jnp.concatenate(
        [dst, jnp.full((E_PAD - E,), GARBAGE, jnp.int32)]
    ).reshape(E_PAD // EB, EB)
    xp = jnp.concatenate([x, jnp.zeros((N_PAD - N, INF), jnp.float32)])
    z16 = jnp.zeros((Z_ROWS, FW), jnp.float32)
    ones16 = jnp.ones((EB, FW), jnp.float32)
    offs = jnp.searchsorted(
        batch, jnp.arange(NGRAPH + 1, dtype=jnp.int32)).astype(jnp.int32)

    d0, d1 = _sc_deg(dstp, ones16, z16)
    g0, g1, g2, g3, dinv = _tc_first(xp, d0, d1, W0)
    gs = (g0, g1, g2, g3)
    for W, b in ((W1, b0), (W2, b1), (W3, b2)):
        a = _sc_agg(*gs, srcp, dstp, z16)
        gs = _tc_layer(*a, *gs, dinv, b.reshape(1, EMB), W)
    a = _sc_agg(*gs, srcp, dstp, z16)
    out, hidden = _tc_final(*a, *gs, dinv, b3.reshape(1, EMB),
                            Wout, bout.reshape(1, 1), offs)
    return (out, hidden)


# speedup vs baseline: 10.1744x; 1.1797x over previous
import functools

import jax
import jax.numpy as jnp
from jax import lax
from jax.experimental import pallas as pl
from jax.experimental.pallas import tpu as pltpu
from jax.experimental.pallas import tpu_sc as plsc

N = 100000
E = 1600000
NGRAPH = 1024
EMB = 64
INF = 9

ROW_BLK = 512
NBLK = 196
N_PAD = NBLK * ROW_BLK
ACC_ROWS = 100368
GARBAGE = N_PAD
Z_ROWS = ACC_ROWS // 16
OUT_ROWS = N_PAD // 16

EB = 128
NSUB = 32
NB = 392
KQ = 8
NQ = NB // KQ
E_PAD = NSUB * NB * EB
FP = 4
FW = 16

_MESH = plsc.VectorSubcoreMesh(core_axis_name="c", subcore_axis_name="s")


def _sc_deg_body(dst_hbm, ones_hbm, zero_hbm, out, acc, didx_v, ones_v,
                 sems):
    c = lax.axis_index("c")
    s = lax.axis_index("s")
    w = c * 16 + s
    pltpu.sync_copy(ones_hbm, ones_v)
    pltpu.sync_copy(zero_hbm, acc.at[pl.ds(s * Z_ROWS, Z_ROWS)])
    plsc.subcore_barrier()

    def bb(q, carry):
        r0 = (w * NQ + q) * KQ
        pltpu.sync_copy(dst_hbm.at[pl.ds(r0, KQ)], didx_v)
        sds = [pltpu.async_copy(ones_v, acc.at[didx_v.at[j]], sems, add=True)
               for j in range(KQ)]
        for d in sds:
            d.wait()
        return carry

    lax.fori_loop(0, NQ, bb, 0)
    plsc.subcore_barrier()

    for ci in range(2):
        @pl.when(c == ci)
        def _(ci=ci):
            pltpu.sync_copy(
                acc.at[pl.ds(s * OUT_ROWS, OUT_ROWS)],
                out.at[pl.ds(s * OUT_ROWS, OUT_ROWS), pl.ds(ci * FW, FW)])


_SC_PARAMS = pltpu.CompilerParams(use_tc_tiling_on_sc=False)

_sc_deg = pl.kernel(
    _sc_deg_body,
    out_type=jax.ShapeDtypeStruct((N_PAD, 128), jnp.float32),
    mesh=_MESH,
    compiler_params=_SC_PARAMS,
    scratch_types=[
        pltpu.VMEM_SHARED((ACC_ROWS, FW), jnp.float32),
        pltpu.VMEM((KQ, EB), jnp.int32),
        pltpu.VMEM((EB, FW), jnp.float32),
        pltpu.SemaphoreType.DMA,
    ],
)


def _sc_agg_body(g0, g1, g2, g3, src_hbm, dst_hbm, zero_hbm, out,
                 acc, idx_v, didx_v, rows_v, sem, sems):
    c = lax.axis_index("c")
    s = lax.axis_index("s")
    w = c * 16 + s
    gs_list = [g0, g1, g2, g3]
    for p in range(FP):
        pltpu.sync_copy(zero_hbm, acc.at[pl.ds(s * Z_ROWS, Z_ROWS)])
        plsc.subcore_barrier()

        def bb(q, carry, p=p):
            r0 = (w * NQ + q) * KQ
            pltpu.sync_copy(src_hbm.at[pl.ds(r0, KQ)], idx_v)
            pltpu.sync_copy(dst_hbm.at[pl.ds(r0, KQ)], didx_v)
            gds = [pltpu.async_copy(gs_list[p].at[idx_v.at[j]],
                                    rows_v.at[j], sem)
                   for j in range(KQ)]
            for d in gds:
                d.wait()
            sds = [pltpu.async_copy(rows_v.at[j], acc.at[didx_v.at[j]],
                                    sems, add=True)
                   for j in range(KQ)]
            for d in sds:
                d.wait()
            return carry

        lax.fori_loop(0, NQ, bb, 0)
        plsc.subcore_barrier()

        for ci in range(2):
            @pl.when(c == ci)
            def _(ci=ci, p=p):
                pltpu.sync_copy(
                    acc.at[pl.ds(s * OUT_ROWS, OUT_ROWS)],
                    out.at[pl.ds(s * OUT_ROWS, OUT_ROWS),
                           pl.ds(ci * EMB + p * FW, FW)])

        plsc.subcore_barrier()


_sc_agg = pl.kernel(
    _sc_agg_body,
    out_type=jax.ShapeDtypeStruct((N_PAD, 128), jnp.float32),
    mesh=_MESH,
    compiler_params=_SC_PARAMS,
    scratch_types=[
        pltpu.VMEM_SHARED((ACC_ROWS, FW), jnp.float32),
        pltpu.VMEM((KQ, EB), jnp.int32),
        pltpu.VMEM((KQ, EB), jnp.int32),
        pltpu.VMEM((KQ, EB, FW), jnp.float32),
        pltpu.SemaphoreType.DMA,
        pltpu.SemaphoreType.DMA,
    ],
)


def _first_body(x_ref, d_ref, w_ref, o0, o1, o2, o3, dinv_ref):
    cnt = d_ref[:, :1] + d_ref[:, FW:FW + 1]
    dinv = lax.rsqrt(cnt + 1.0)
    g = jnp.dot(x_ref[...], w_ref[...], preferred_element_type=jnp.float32)
    gs = dinv * g
    for p, o in enumerate((o0, o1, o2, o3)):
        o[...] = gs[:, p * FW:(p + 1) * FW]
    dinv_ref[...] = dinv


_tc_first = pl.pallas_call(
    _first_body,
    grid=(NBLK,),
    in_specs=[
        pl.BlockSpec((ROW_BLK, INF), lambda i: (i, 0)),
        pl.BlockSpec((ROW_BLK, 128), lambda i: (i, 0)),
        pl.BlockSpec((INF, EMB), lambda i: (0, 0)),
    ],
    out_specs=[pl.BlockSpec((ROW_BLK, FW), lambda i: (i, 0))] * 4
    + [pl.BlockSpec((ROW_BLK, 1), lambda i: (i, 0))],
    out_shape=[jax.ShapeDtypeStruct((N_PAD, FW), jnp.float32)] * 4
    + [jax.ShapeDtypeStruct((N_PAD, 1), jnp.float32)],
)


def _layer_body(a_ref, g0, g1, g2, g3, dinv_ref, b_ref, w_ref,
                o0, o1, o2, o3):
    A = a_ref[:, :EMB] + a_ref[:, EMB:]
    G = jnp.concatenate([g0[...], g1[...], g2[...], g3[...]], axis=1)
    dinv = dinv_ref[...]
    h = jnp.tanh(dinv * (A + G) + b_ref[...])
    gs = dinv * jnp.dot(h, w_ref[...], preferred_element_type=jnp.float32)
    for p, o in enumerate((o0, o1, o2, o3)):
        o[...] = gs[:, p * FW:(p + 1) * FW]


_tc_layer = pl.pallas_call(
    _layer_body,
    grid=(NBLK,),
    in_specs=[pl.BlockSpec((ROW_BLK, 128), lambda i: (i, 0))]
    + [pl.BlockSpec((ROW_BLK, FW), lambda i: (i, 0))] * 4
    + [
        pl.BlockSpec((ROW_BLK, 1), lambda i: (i, 0)),
        pl.BlockSpec((1, EMB), lambda i: (0, 0)),
        pl.BlockSpec((EMB, EMB), lambda i: (0, 0)),
    ],
    out_specs=[pl.BlockSpec((ROW_BLK, FW), lambda i: (i, 0))] * 4,
    out_shape=[jax.ShapeDtypeStruct((N_PAD, FW), jnp.float32)] * 4,
)


def _final_body(a_ref, g0, g1, g2, g3, dinv_ref, b_ref, wout_ref, bout_ref,
                offs_ref, out_ref, hid_ref, h4_ref):
    i = pl.program_id(0)
    A = a_ref[:, :EMB] + a_ref[:, EMB:]
    G = jnp.concatenate([g0[...], g1[...], g2[...], g3[...]], axis=1)
    h = jnp.tanh(dinv_ref[...] * (A + G) + b_ref[...])
    h4_ref[pl.ds(i * ROW_BLK, ROW_BLK), :] = h

    @pl.when(i == NBLK - 1)
    def _():
        def pool(gi, carry):
            s0 = offs_ref[gi]
            e0 = offs_ref[gi + 1]
            p0 = jnp.bitwise_and(s0, -8)

            def cond(c):
                return c[0] < e0

            def bodyw(c):
                p, mx, sm = c
                rows = h4_ref[pl.ds(p, 128), :]
                ids = p + lax.broadcasted_iota(jnp.int32, (128, 1), 0)
                m = (ids >= s0) & (ids < e0)
                mx = jnp.maximum(mx, jnp.where(m, rows, -jnp.inf))
                sm = sm + jnp.where(m, rows, 0.0)
                return (p + 128, mx, sm)

            neg = jnp.full((128, EMB), -jnp.inf, jnp.float32)
            zer = jnp.zeros((128, EMB), jnp.float32)
            _, mx, sm = lax.while_loop(cond, bodyw, (p0, neg, zer))
            gmx = jnp.max(mx, axis=0, keepdims=True)
            gsm = jnp.sum(sm, axis=0, keepdims=True)
            cntf = jnp.maximum((e0 - s0).astype(jnp.float32), 1.0)
            hid_ref[pl.ds(gi, 1), :] = jnp.concatenate([gmx, gsm / cntf],
                                                       axis=1)
            return carry

        lax.fori_loop(0, NGRAPH, pool, 0)
        hid = hid_ref[...]
        out_ref[...] = jnp.dot(hid, wout_ref[...],
                               preferred_element_type=jnp.float32) + bout_ref[...]


_tc_final = pl.pallas_call(
    _final_body,
    grid=(NBLK,),
    in_specs=[pl.BlockSpec((ROW_BLK, 128), lambda i: (i, 0))]
    + [pl.BlockSpec((ROW_BLK, FW), lambda i: (i, 0))] * 4
    + [
        pl.BlockSpec((ROW_BLK, 1), lambda i: (i, 0)),
        pl.BlockSpec((1, EMB), lambda i: (0, 0)),
        pl.BlockSpec((2 * EMB, 1), lambda i: (0, 0)),
        pl.BlockSpec((1, 1), lambda i: (0, 0)),
        pl.BlockSpec(memory_space=pltpu.SMEM),
    ],
    out_specs=[
        pl.BlockSpec((NGRAPH, 1), lambda i: (0, 0)),
        pl.BlockSpec((NGRAPH, 2 * EMB), lambda i: (0, 0)),
    ],
    out_shape=[
        jax.ShapeDtypeStruct((NGRAPH, 1), jnp.float32),
        jax.ShapeDtypeStruct((NGRAPH, 2 * EMB), jnp.float32),
    ],
    scratch_shapes=[pltpu.VMEM((N_PAD, EMB), jnp.float32)],
)


@jax.jit
def kernel(x, edge_index, batch_index, W0, b0, W1, b1, W2, b2, W3, b3,
           Wout, bout):
    src = edge_index[0].astype(jnp.int32)
    dst = edge_index[1].astype(jnp.int32)
    batch = batch_index.astype(jnp.int32)

    srcp = jnp.concatenate(
        [src, jnp.zeros((E_PAD - E,), jnp.int32)]).reshape(E_PAD // EB, EB)
    dstp = jnp.concatenate(
        [dst, jnp.full((E_PAD - E,), GARBAGE, jnp.int32)]
    ).reshape(E_PAD // EB, EB)
    xp = jnp.concatenate([x, jnp.zeros((N_PAD - N, INF), jnp.float32)])
    z16 = jnp.zeros((Z_ROWS, FW), jnp.float32)
    ones16 = jnp.ones((EB, FW), jnp.float32)
    offs = jnp.searchsorted(
        batch, jnp.arange(NGRAPH + 1, dtype=jnp.int32)).astype(jnp.int32)

    d = _sc_deg(dstp, ones16, z16)
    g0, g1, g2, g3, dinv = _tc_first(xp, d, W0)
    gs = (g0, g1, g2, g3)
    for W, b in ((W1, b0), (W2, b1), (W3, b2)):
        a = _sc_agg(*gs, srcp, dstp, z16)
        gs = _tc_layer(a, *gs, dinv, b.reshape(1, EMB), W)
    a = _sc_agg(*gs, srcp, dstp, z16)
    out, hidden = _tc_final(a, *gs, dinv, b3.reshape(1, EMB),
                            Wout, bout.reshape(1, 1), offs)
    return (out, hidden)
